# baseline (speedup 1.0000x reference)
import functools

import numpy as np
import jax
import jax.numpy as jnp
from jax import lax
from jax.experimental import pallas as pl
from jax.experimental.pallas import tpu as pltpu
from jax.experimental.pallas import tpu_sc as plsc

B, H, N, F = 8, 12, 576, 768
G = 16
HB = 2
L = 16
NC, NS = 2, 16
NEG = np.float32(-3.0e38)


def _reduce_body(a_ref, o_ref):
    h = pl.program_id(1)
    part = jnp.sum(a_ref[...], axis=(1, 2))[:, None, :]

    @pl.when(h == 0)
    def _():
        o_ref[...] = part

    @pl.when(h != 0)
    def _():
        o_ref[...] += part


def _scores(attn):
    return pl.pallas_call(
        _reduce_body,
        grid=(B, H // HB),
        in_specs=[pl.BlockSpec((1, HB, N, N), lambda b, h: (b, h, 0, 0))],
        out_specs=pl.BlockSpec((1, 1, N), lambda b, h: (b, 0, 0)),
        out_shape=jax.ShapeDtypeStruct((B, 1, N), jnp.float32),
    )(attn).reshape(B, N)


def _topk_gather(scores, x2):
    mesh = plsc.VectorSubcoreMesh(core_axis_name="c", subcore_axis_name="s")

    @functools.partial(
        pl.kernel,
        out_type=jax.ShapeDtypeStruct((B * G, F), jnp.float32),
        mesh=mesh,
        scratch_types=[
            pltpu.VMEM((N,), jnp.float32),
            pltpu.VMEM((G,), jnp.int32),
            pltpu.VMEM((G, F), jnp.float32),
            pltpu.SemaphoreType.DMA,
        ],
    )
    def k(scores_hbm, x_hbm, out_hbm, s_v, idx_v, rows_v, sem):
        wid = lax.axis_index("s") * NC + lax.axis_index("c")

        @pl.when(wid < B)
        def _():
            b = wid
            pltpu.sync_copy(scores_hbm.at[b], s_v)
            lanes = lax.iota(jnp.int32, L)

            def outer(k_i, topk):
                def scan(j, c):
                    bv, bi = c
                    v = s_v[pl.ds(j * L, L)]
                    take = v > bv
                    return (jnp.where(take, v, bv),
                            jnp.where(take, j * L + lanes, bi))

                bv, bi = lax.fori_loop(
                    0, N // L, scan,
                    (jnp.full((L,), NEG, jnp.float32),
                     jnp.zeros((L,), jnp.int32)))
                best, besti = bv[0], bi[0]
                for i in range(1, L):
                    vi, ni = bv[i], bi[i]
                    upd = (vi > best) | ((vi == best) & (ni < besti))
                    best = jnp.where(upd, vi, best)
                    besti = jnp.where(upd, ni, besti)
                cb = (besti // L) * L
                cur = s_v[pl.ds(cb, L)]
                s_v[pl.ds(cb, L)] = jnp.where(lanes == besti - cb, NEG, cur)
                return jnp.where(lanes == k_i, besti, topk)

            topk = lax.fori_loop(0, G, outer, jnp.zeros((L,), jnp.int32))
            idx_v[...] = topk + b * N
            pltpu.async_copy(x_hbm.at[idx_v], rows_v, sem).wait()
            pltpu.sync_copy(rows_v, out_hbm.at[pl.ds(b * G, G)])

    return k(scores, x2)


def kernel(x, attn_weights):
    scores = _scores(attn_weights)
    out = _topk_gather(scores, x.reshape(B * N, F))
    return out.reshape(B, G, F)


# speedup vs baseline: 1.1467x; 1.1467x over previous
import functools

import numpy as np
import jax
import jax.numpy as jnp
from jax import lax
from jax.experimental import pallas as pl
from jax.experimental.pallas import tpu as pltpu
from jax.experimental.pallas import tpu_sc as plsc

B, H, N, F = 8, 12, 576, 768
G = 16
HB = 6
L = 16
NC, NS = 2, 16
NEG = np.float32(-3.0e38)


def _reduce_body(a_ref, o_ref):
    h = pl.program_id(1)
    part = jnp.sum(a_ref[...], axis=(1, 2))[:, None, :]

    @pl.when(h == 0)
    def _():
        o_ref[...] = part

    @pl.when(h != 0)
    def _():
        o_ref[...] += part


def _scores(attn):
    return pl.pallas_call(
        _reduce_body,
        grid=(B, H // HB),
        in_specs=[pl.BlockSpec((1, HB, N, N), lambda b, h: (b, h, 0, 0))],
        out_specs=pl.BlockSpec((1, 1, N), lambda b, h: (b, 0, 0)),
        out_shape=jax.ShapeDtypeStruct((B, 1, N), jnp.float32),
    )(attn).reshape(B, N)


def _topk_gather(scores, x2):
    mesh = plsc.VectorSubcoreMesh(core_axis_name="c", subcore_axis_name="s")

    @functools.partial(
        pl.kernel,
        out_type=jax.ShapeDtypeStruct((B * G, F), jnp.float32),
        mesh=mesh,
        scratch_types=[
            pltpu.VMEM((N,), jnp.float32),
            pltpu.VMEM((G,), jnp.int32),
            pltpu.VMEM((G, F), jnp.float32),
            pltpu.SemaphoreType.DMA,
        ],
    )
    def k(scores_hbm, x_hbm, out_hbm, s_v, idx_v, rows_v, sem):
        wid = lax.axis_index("s") * NC + lax.axis_index("c")

        @pl.when(wid < B)
        def _():
            b = wid
            pltpu.sync_copy(scores_hbm.at[b], s_v)
            lanes = lax.iota(jnp.int32, L)

            def outer(k_i, topk):
                def scan(j, c):
                    bv, bi = c
                    v = s_v[pl.ds(j * L, L)]
                    take = v > bv
                    return (jnp.where(take, v, bv),
                            jnp.where(take, j * L + lanes, bi))

                bv, bi = lax.fori_loop(
                    0, N // L, scan,
                    (jnp.full((L,), NEG, jnp.float32),
                     jnp.zeros((L,), jnp.int32)))
                best, besti = bv[0], bi[0]
                for i in range(1, L):
                    vi, ni = bv[i], bi[i]
                    upd = (vi > best) | ((vi == best) & (ni < besti))
                    best = jnp.where(upd, vi, best)
                    besti = jnp.where(upd, ni, besti)
                cb = (besti // L) * L
                cur = s_v[pl.ds(cb, L)]
                s_v[pl.ds(cb, L)] = jnp.where(lanes == besti - cb, NEG, cur)
                return jnp.where(lanes == k_i, besti, topk)

            topk = lax.fori_loop(0, G, outer, jnp.zeros((L,), jnp.int32))
            idx_v[...] = topk + b * N
            pltpu.async_copy(x_hbm.at[idx_v], rows_v, sem).wait()
            pltpu.sync_copy(rows_v, out_hbm.at[pl.ds(b * G, G)])

    return k(scores, x2)


def kernel(x, attn_weights):
    scores = _scores(attn_weights)
    out = _topk_gather(scores, x.reshape(B * N, F))
    return out.reshape(B, G, F)


# speedup vs baseline: 1.1632x; 1.0144x over previous
import functools

import numpy as np
import jax
import jax.numpy as jnp
from jax import lax
from jax.experimental import pallas as pl
from jax.experimental.pallas import tpu as pltpu
from jax.experimental.pallas import tpu_sc as plsc

B, H, N, F = 8, 12, 576, 768
G = 16
HB = 6
L = 16
NC, NS = 2, 16
NEG = np.float32(-3.0e38)


CH = 2
NBUF = 8


def _scores(attn):
    nch = B * H // CH
    hpc = H // CH
    attn4 = attn.reshape(nch, CH, N, N)

    def body(a_hbm, o_ref, bufs, sems):
        def start(i):
            slot = i % NBUF
            pltpu.make_async_copy(a_hbm.at[i], bufs.at[slot],
                                  sems.at[slot]).start()

        for i in range(NBUF):
            start(i)
        for b in range(B):
            acc = jnp.zeros((N,), jnp.float32)
            for hh in range(hpc):
                i = b * hpc + hh
                slot = i % NBUF
                pltpu.make_async_copy(a_hbm.at[i], bufs.at[slot],
                                      sems.at[slot]).wait()
                acc = acc + jnp.sum(bufs[slot], axis=(0, 1))
                if i + NBUF < nch:
                    start(i + NBUF)
            o_ref[b] = acc

    return pl.pallas_call(
        body,
        in_specs=[pl.BlockSpec(memory_space=pltpu.HBM)],
        out_specs=pl.BlockSpec(memory_space=pltpu.VMEM),
        out_shape=jax.ShapeDtypeStruct((B, N), jnp.float32),
        scratch_shapes=[pltpu.VMEM((NBUF, CH, N, N), jnp.float32),
                        pltpu.SemaphoreType.DMA((NBUF,))],
    )(attn4)


def _topk_gather(scores, x2):
    mesh = plsc.VectorSubcoreMesh(core_axis_name="c", subcore_axis_name="s")

    @functools.partial(
        pl.kernel,
        out_type=jax.ShapeDtypeStruct((B * G, F), jnp.float32),
        mesh=mesh,
        scratch_types=[
            pltpu.VMEM((N,), jnp.float32),
            pltpu.VMEM((G,), jnp.int32),
            pltpu.VMEM((G, F), jnp.float32),
            pltpu.SemaphoreType.DMA,
        ],
    )
    def k(scores_hbm, x_hbm, out_hbm, s_v, idx_v, rows_v, sem):
        wid = lax.axis_index("s") * NC + lax.axis_index("c")

        @pl.when(wid < B)
        def _():
            b = wid
            pltpu.sync_copy(scores_hbm.at[b], s_v)
            lanes = lax.iota(jnp.int32, L)

            def outer(k_i, topk):
                def scan(j, c):
                    bv, bi = c
                    v = s_v[pl.ds(j * L, L)]
                    take = v > bv
                    return (jnp.where(take, v, bv),
                            jnp.where(take, j * L + lanes, bi))

                bv, bi = lax.fori_loop(
                    0, N // L, scan,
                    (jnp.full((L,), NEG, jnp.float32),
                     jnp.zeros((L,), jnp.int32)))
                best, besti = bv[0], bi[0]
                for i in range(1, L):
                    vi, ni = bv[i], bi[i]
                    upd = (vi > best) | ((vi == best) & (ni < besti))
                    best = jnp.where(upd, vi, best)
                    besti = jnp.where(upd, ni, besti)
                cb = (besti // L) * L
                cur = s_v[pl.ds(cb, L)]
                s_v[pl.ds(cb, L)] = jnp.where(lanes == besti - cb, NEG, cur)
                return jnp.where(lanes == k_i, besti, topk)

            topk = lax.fori_loop(0, G, outer, jnp.zeros((L,), jnp.int32))
            idx_v[...] = topk + b * N
            pltpu.async_copy(x_hbm.at[idx_v], rows_v, sem).wait()
            pltpu.sync_copy(rows_v, out_hbm.at[pl.ds(b * G, G)])

    return k(scores, x2)


def kernel(x, attn_weights):
    scores = _scores(attn_weights)
    out = _topk_gather(scores, x.reshape(B * N, F))
    return out.reshape(B, G, F)
